# Initial kernel scaffold; baseline (speedup 1.0000x reference)
import jax
import jax.numpy as jnp
from jax.experimental import pallas as pl


def kernel(inputs, router_w, W1, b1, ln_g, ln_b, W2, b2):
    raise NotImplementedError("write your pallas kernel here")


# speedup vs baseline: 2.0467x; 2.0467x over previous
import functools

import jax
import jax.numpy as jnp
from jax import lax
from jax.experimental import pallas as pl
from jax.experimental.pallas import tpu as pltpu
from jax.experimental.pallas import tpu_sc as plsc

_pallas_call = pl.pallas_call

N = 8192
D = 1024
F = 2048
E = 8
K = 1024
EPS_LN = 1e-05
MT = 512
MTILES = K // MT


_ROUTER_DTYPE = jnp.bfloat16


def _router_body(x_ref, rw_ref, logits_ref, probsT_ref):
    xb = x_ref[...].astype(_ROUTER_DTYPE)
    rw = rw_ref[...].astype(_ROUTER_DTYPE)
    logits_ref[...] = jnp.dot(xb, rw, preferred_element_type=jnp.float32)
    lT = lax.dot_general(rw, xb, (((0,), (1,)), ((), ())),
                         preferred_element_type=jnp.float32)
    m = jnp.max(lT, axis=0, keepdims=True)
    ex = jnp.exp(lT - m)
    probsT_ref[...] = ex / jnp.sum(ex, axis=0, keepdims=True)


def _thresh_body(probsT_ref, thr_ref, cnt_ref):
    bits = lax.bitcast_convert_type(probsT_ref[...], jnp.int32)

    def body(_, lohi):
        lo, hi = lohi
        mid = (lo + hi) // 2
        cnt = jnp.sum((bits >= mid).astype(jnp.int32), axis=1, keepdims=True)
        ge = cnt >= K
        return jnp.where(ge, mid, lo), jnp.where(ge, hi, mid)

    lo0 = jnp.zeros((E, 1), jnp.int32)
    hi0 = jnp.full((E, 1), 0x40000000, jnp.int32)
    thr, _ = lax.fori_loop(0, 31, body, (lo0, hi0))
    c = jnp.sum((bits >= (thr + 1)).astype(jnp.int32), axis=1, keepdims=True)
    thr_ref[...] = jnp.broadcast_to(thr, (E, 128))
    cnt_ref[...] = jnp.broadcast_to(c, (E, 128))


def _ffn_body(xg_ref, w1_ref, b1_ref, g_ref, bln_ref, w2_ref, b2_ref,
              wts_ref, out_ref):
    xi = xg_ref[...].astype(jnp.bfloat16)
    w1 = w1_ref[0].astype(jnp.bfloat16)
    h = jnp.dot(xi, w1, preferred_element_type=jnp.float32) + b1_ref[0]
    h = 0.5 * h * (1.0 + lax.erf(h * 0.7071067811865476))
    mu = jnp.mean(h, axis=-1, keepdims=True)
    var = jnp.mean(jnp.square(h - mu), axis=-1, keepdims=True)
    h = (h - mu) * lax.rsqrt(var + EPS_LN) * g_ref[0] + bln_ref[0]
    w2 = w2_ref[0].astype(jnp.bfloat16)
    out = jnp.dot(h.astype(jnp.bfloat16), w2,
                  preferred_element_type=jnp.float32) + b2_ref[0]
    wl = wts_ref[0]
    rows = lax.broadcasted_iota(jnp.int32, (MT, MT), 0)
    cols = lax.broadcasted_iota(jnp.int32, (MT, MT), 1)
    wcol = jnp.sum(jnp.where(rows == cols, wl, 0.0), axis=1, keepdims=True)
    out = out * wcol
    out_ref[0] = out[:, :D // 2]
    out_ref[1] = out[:, D // 2:]


def _router_probs(x, router_w):
    TB = 1024
    return _pallas_call(
        _router_body,
        grid=(N // TB,),
        in_specs=[pl.BlockSpec((TB, D), lambda i: (i, 0)),
                  pl.BlockSpec((D, E), lambda i: (0, 0))],
        out_specs=[pl.BlockSpec((TB, E), lambda i: (i, 0)),
                   pl.BlockSpec((E, TB), lambda i: (0, i))],
        out_shape=[jax.ShapeDtypeStruct((N, E), jnp.float32),
                   jax.ShapeDtypeStruct((E, N), jnp.float32)],
    )(x, router_w)


def _thresholds(probsT):
    return _pallas_call(
        _thresh_body,
        in_specs=[pl.BlockSpec((E, N), lambda: (0, 0))],
        out_specs=[pl.BlockSpec((E, 128), lambda: (0, 0)),
                   pl.BlockSpec((E, 128), lambda: (0, 0))],
        out_shape=[jax.ShapeDtypeStruct((E, 128), jnp.int32),
                   jax.ShapeDtypeStruct((E, 128), jnp.int32)],
    )(probsT)


def _ffn(xg, W1, b1, ln_g, ln_b, W2, b2, wts):
    b1r = b1.reshape(E, 1, F)
    gr = ln_g.reshape(E, 1, F)
    br = ln_b.reshape(E, 1, F)
    b2r = b2.reshape(E, 1, D)
    wtsr = wts.reshape(E, 1, K)
    return _pallas_call(
        _ffn_body,
        grid=(E, MTILES),
        in_specs=[pl.BlockSpec((MT, D), lambda e, m: (e * MTILES + m, 0)),
                  pl.BlockSpec((1, D, F), lambda e, m: (e, 0, 0)),
                  pl.BlockSpec((1, 1, F), lambda e, m: (e, 0, 0)),
                  pl.BlockSpec((1, 1, F), lambda e, m: (e, 0, 0)),
                  pl.BlockSpec((1, 1, F), lambda e, m: (e, 0, 0)),
                  pl.BlockSpec((1, F, D), lambda e, m: (e, 0, 0)),
                  pl.BlockSpec((1, 1, D), lambda e, m: (e, 0, 0)),
                  pl.BlockSpec((1, 1, MT), lambda e, m: (e, 0, m))],
        out_specs=pl.BlockSpec((2, MT, D // 2),
                               lambda e, m: (0, e * MTILES + m, 0)),
        out_shape=jax.ShapeDtypeStruct((2, N, D // 2), jnp.float32),
    )(xg, W1, b1r, gr, br, W2, b2r, wtsr)


def _select_glue(probsT, thr_row, cnt_row):
    bits = lax.bitcast_convert_type(probsT, jnp.int32)
    thr = thr_row[:, :1]
    need = K - cnt_row[:, :1]
    gt = bits > thr
    eq = bits == thr
    eq_rank = jnp.cumsum(eq.astype(jnp.int32), axis=1)
    sel = gt | (eq & (eq_rank <= need))
    order = jnp.argsort(jnp.where(sel, 0, 1), axis=1, stable=True)
    idx = order[:, :K]
    wts = jnp.take_along_axis(probsT, idx, axis=1)
    return idx.astype(jnp.int32), wts


def kernel(inputs, router_w, W1, b1, ln_g, ln_b, W2, b2):
    B, S, _ = inputs.shape
    x = inputs.reshape(N, D)
    logits, probsT = _router_probs(x, router_w)
    thr_row, cnt_row = _thresholds(probsT)
    idx, wts = _select_glue(probsT, thr_row, cnt_row)
    idx_flat = idx.reshape(-1)
    xg = jnp.take(x, idx_flat, axis=0)
    out01 = _ffn(xg, W1, b1, ln_g, ln_b, W2, b2, wts)
    out_rows = jnp.concatenate([out01[0], out01[1]], axis=-1)
    results = jnp.zeros((N, D), jnp.float32).at[idx_flat].add(out_rows)
    return results.reshape(B, S, D), logits.reshape(B, S, E)


# speedup vs baseline: 2.2672x; 1.1077x over previous
import functools

import jax
import jax.numpy as jnp
from jax import lax
from jax.experimental import pallas as pl
from jax.experimental.pallas import tpu as pltpu
from jax.experimental.pallas import tpu_sc as plsc

_pallas_call = pl.pallas_call

N = 8192
D = 1024
F = 2048
E = 8
K = 1024
EPS_LN = 1e-05
MT = 512
MTILES = K // MT


_ROUTER_DTYPE = jnp.bfloat16


def _router_body(x_ref, rw_ref, logits_ref, probsT_ref):
    xb = x_ref[...].astype(_ROUTER_DTYPE)
    rw = rw_ref[...].astype(_ROUTER_DTYPE)
    logits_ref[...] = jnp.dot(xb, rw, preferred_element_type=jnp.float32)
    lT = lax.dot_general(rw, xb, (((0,), (1,)), ((), ())),
                         preferred_element_type=jnp.float32)
    m = jnp.max(lT, axis=0, keepdims=True)
    ex = jnp.exp(lT - m)
    probsT_ref[...] = ex / jnp.sum(ex, axis=0, keepdims=True)


def _thresh_body(probsT_ref, thr_ref, cnt_ref):
    bits = lax.bitcast_convert_type(probsT_ref[...], jnp.int32)

    def body(_, lohi):
        lo, hi = lohi
        mid = (lo + hi) // 2
        cnt = jnp.sum((bits >= mid).astype(jnp.int32), axis=1, keepdims=True)
        ge = cnt >= K
        return jnp.where(ge, mid, lo), jnp.where(ge, hi, mid)

    lo0 = jnp.zeros((E, 1), jnp.int32)
    hi0 = jnp.full((E, 1), 0x40000000, jnp.int32)
    thr, _ = lax.fori_loop(0, 31, body, (lo0, hi0))
    c = jnp.sum((bits >= (thr + 1)).astype(jnp.int32), axis=1, keepdims=True)
    thr_f = lax.bitcast_convert_type(thr, jnp.float32)
    thr_ref[...] = jnp.broadcast_to(thr_f, (E, 128))
    cnt_ref[...] = jnp.broadcast_to(c, (E, 128))


def _ffn_body(xg_ref, w1_ref, b1_ref, g_ref, bln_ref, w2_ref, b2_ref,
              wts_ref, out_ref):
    xi = xg_ref[...].astype(jnp.bfloat16)
    w1 = w1_ref[0].astype(jnp.bfloat16)
    h = jnp.dot(xi, w1, preferred_element_type=jnp.float32) + b1_ref[0]
    h = 0.5 * h * (1.0 + lax.erf(h * 0.7071067811865476))
    mu = jnp.mean(h, axis=-1, keepdims=True)
    var = jnp.mean(jnp.square(h - mu), axis=-1, keepdims=True)
    h = (h - mu) * lax.rsqrt(var + EPS_LN) * g_ref[0] + bln_ref[0]
    w2 = w2_ref[0].astype(jnp.bfloat16)
    out = jnp.dot(h.astype(jnp.bfloat16), w2,
                  preferred_element_type=jnp.float32) + b2_ref[0]
    wl = wts_ref[0]
    rows = lax.broadcasted_iota(jnp.int32, (MT, MT), 0)
    cols = lax.broadcasted_iota(jnp.int32, (MT, MT), 1)
    wcol = jnp.sum(jnp.where(rows == cols, wl, 0.0), axis=1, keepdims=True)
    out = out * wcol
    out_ref[0] = out[:, :D // 2]
    out_ref[1] = out[:, D // 2:]


def _router_probs(x, router_w):
    TB = 1024
    return _pallas_call(
        _router_body,
        grid=(N // TB,),
        in_specs=[pl.BlockSpec((TB, D), lambda i: (i, 0)),
                  pl.BlockSpec((D, E), lambda i: (0, 0))],
        out_specs=[pl.BlockSpec((TB, E), lambda i: (i, 0)),
                   pl.BlockSpec((E, TB), lambda i: (0, i))],
        out_shape=[jax.ShapeDtypeStruct((N, E), jnp.float32),
                   jax.ShapeDtypeStruct((E, N), jnp.float32)],
    )(x, router_w)


def _thresholds(probsT):
    return _pallas_call(
        _thresh_body,
        in_specs=[pl.BlockSpec((E, N), lambda: (0, 0))],
        out_specs=[pl.BlockSpec((E, 128), lambda: (0, 0)),
                   pl.BlockSpec((E, 128), lambda: (0, 0))],
        out_shape=[jax.ShapeDtypeStruct((E, 128), jnp.float32),
                   jax.ShapeDtypeStruct((E, 128), jnp.int32)],
    )(probsT)


def _ffn(xg, W1, b1, ln_g, ln_b, W2, b2, wts):
    b1r = b1.reshape(E, 1, F)
    gr = ln_g.reshape(E, 1, F)
    br = ln_b.reshape(E, 1, F)
    b2r = b2.reshape(E, 1, D)
    wtsr = wts.reshape(E, 1, K)
    return _pallas_call(
        _ffn_body,
        grid=(E, MTILES),
        in_specs=[pl.BlockSpec((MT, D), lambda e, m: (e * MTILES + m, 0)),
                  pl.BlockSpec((1, D, F), lambda e, m: (e, 0, 0)),
                  pl.BlockSpec((1, 1, F), lambda e, m: (e, 0, 0)),
                  pl.BlockSpec((1, 1, F), lambda e, m: (e, 0, 0)),
                  pl.BlockSpec((1, 1, F), lambda e, m: (e, 0, 0)),
                  pl.BlockSpec((1, F, D), lambda e, m: (e, 0, 0)),
                  pl.BlockSpec((1, 1, D), lambda e, m: (e, 0, 0)),
                  pl.BlockSpec((1, 1, MT), lambda e, m: (e, 0, m))],
        out_specs=pl.BlockSpec((2, MT, D // 2),
                               lambda e, m: (0, e * MTILES + m, 0)),
        out_shape=jax.ShapeDtypeStruct((2, N, D // 2), jnp.float32),
    )(xg, W1, b1r, gr, br, W2, b2r, wtsr)


_SC_MESH = plsc.VectorSubcoreMesh(core_axis_name="c", subcore_axis_name="s")


def _compact_body(probsT_hbm, thr_hbm, cnt_hbm, idx_out, wts_out,
                  pcol, thrv, cntv, idxv, wtsv):
    c = lax.axis_index("c")
    s = lax.axis_index("s")
    wid = s * 2 + c

    @pl.when(wid < E)
    def _():
        e = wid
        pltpu.sync_copy(probsT_hbm.at[e], pcol)
        pltpu.sync_copy(thr_hbm.at[e], thrv)
        pltpu.sync_copy(cnt_hbm.at[e], cntv)
        thr_vec = thrv[pl.ds(0, 16)]
        need_vec = K - cntv[pl.ds(0, 16)]

        def body(i, carry):
            off, eqseen = carry
            v = pcol[pl.ds(i * 16, 16)]
            gt = v > thr_vec
            eq = v == thr_vec
            eqc = plsc.cumsum(eq.astype(jnp.int32))
            sel = gt | (eq & ((eqc + eqseen) <= need_vec))
            selc = plsc.cumsum(sel.astype(jnp.int32))
            pos = jnp.where(sel, off + selc - 1, 0)
            tok = lax.iota(jnp.int32, 16) + i * 16
            plsc.store_scatter(idxv, [pos], tok, mask=sel)
            plsc.store_scatter(wtsv, [pos], v, mask=sel)
            nsel = jnp.sum(sel.astype(jnp.int32))
            neq = jnp.sum(eq.astype(jnp.int32))
            return off + nsel, eqseen + neq

        lax.fori_loop(0, N // 16, body, (jnp.int32(0), jnp.int32(0)))
        pltpu.sync_copy(idxv.at[pl.ds(0, K)], idx_out.at[e])
        pltpu.sync_copy(wtsv.at[pl.ds(0, K)], wts_out.at[e])


@functools.partial(
    pl.kernel, mesh=_SC_MESH,
    compiler_params=pltpu.CompilerParams(needs_layout_passes=False),
    out_type=[jax.ShapeDtypeStruct((E, K), jnp.int32),
              jax.ShapeDtypeStruct((E, K), jnp.float32)],
    scratch_types=[pltpu.VMEM((N,), jnp.float32),
                   pltpu.VMEM((128,), jnp.float32),
                   pltpu.VMEM((128,), jnp.int32),
                   pltpu.VMEM((K + 16,), jnp.int32),
                   pltpu.VMEM((K + 16,), jnp.float32)])
def _compact_sc(probsT_hbm, thr_hbm, cnt_hbm, idx_out, wts_out,
                pcol, thrv, cntv, idxv, wtsv):
    _compact_body(probsT_hbm, thr_hbm, cnt_hbm, idx_out, wts_out,
                  pcol, thrv, cntv, idxv, wtsv)


_GC = 64


def _gather_body(x_hbm, idx_hbm, xg_out, idxv, rowsv, sem):
    wid = lax.axis_index("s") * 2 + lax.axis_index("c")
    rows_per = N // 32

    def chunk(ci, _):
        b = wid * rows_per + ci * _GC
        pltpu.sync_copy(idx_hbm.at[pl.ds(b, _GC)], idxv)
        pltpu.async_copy(x_hbm.at[idxv], rowsv, sem).wait()
        pltpu.sync_copy(rowsv, xg_out.at[pl.ds(b, _GC)])
        return 0

    lax.fori_loop(0, rows_per // _GC, chunk, 0)


@functools.partial(
    pl.kernel, mesh=_SC_MESH,
    compiler_params=pltpu.CompilerParams(needs_layout_passes=False),
    out_type=jax.ShapeDtypeStruct((N, D), jnp.float32),
    scratch_types=[pltpu.VMEM((_GC,), jnp.int32),
                   pltpu.VMEM((_GC, D), jnp.float32),
                   pltpu.SemaphoreType.DMA])
def _gather_sc(x_hbm, idx_hbm, xg_out, idxv, rowsv, sem):
    _gather_body(x_hbm, idx_hbm, xg_out, idxv, rowsv, sem)


def _scatter_body(outg2_hbm, idx_hbm, res_out, idxv, tgtv, accv, srcv, sem):
    h = lax.axis_index("c")
    w = lax.axis_index("s")
    DH = D // 2
    zero16 = jnp.zeros((16,), jnp.float32)

    def zrow(r, _):
        for m in range(DH // 16):
            accv[r, pl.ds(m * 16, 16)] = zero16
        return 0

    lax.fori_loop(0, 64, zrow, 0)
    for zc in range(8):
        j0 = w * 512 + zc * 64
        for m in range(4):
            tgtv[pl.ds(m * 16, 16)] = (lax.iota(jnp.int32, 16)
                                       + (j0 + m * 16)) * 2 + h
        pltpu.async_copy(accv, res_out.at[tgtv], sem).wait()
    plsc.subcore_barrier()
    for e in range(E):
        base = e * K + w * 64
        pltpu.sync_copy(idx_hbm.at[pl.ds(base, 64)], idxv)
        for m in range(4):
            tgtv[pl.ds(m * 16, 16)] = idxv[pl.ds(m * 16, 16)] * 2 + h
        pltpu.async_copy(res_out.at[tgtv], accv, sem).wait()
        pltpu.sync_copy(outg2_hbm.at[pl.ds(h * N + base, 64)], srcv)

        def addrow(r, _):
            for m in range(DH // 16):
                sl = pl.ds(m * 16, 16)
                accv[r, sl] = accv[r, sl] + srcv[r, sl]
            return 0

        lax.fori_loop(0, 64, addrow, 0)
        pltpu.async_copy(accv, res_out.at[tgtv], sem).wait()
        plsc.subcore_barrier()


@functools.partial(
    pl.kernel, mesh=_SC_MESH,
    compiler_params=pltpu.CompilerParams(needs_layout_passes=False),
    out_type=jax.ShapeDtypeStruct((2 * N, D // 2), jnp.float32),
    scratch_types=[pltpu.VMEM((64,), jnp.int32),
                   pltpu.VMEM((64,), jnp.int32),
                   pltpu.VMEM((64, D // 2), jnp.float32),
                   pltpu.VMEM((64, D // 2), jnp.float32),
                   pltpu.SemaphoreType.DMA])
def _scatter_sc(outg2_hbm, idx_hbm, res_out, idxv, tgtv, accv, srcv, sem):
    _scatter_body(outg2_hbm, idx_hbm, res_out, idxv, tgtv, accv, srcv, sem)


def _select_glue(probsT, thr_row, cnt_row):
    bits = lax.bitcast_convert_type(probsT, jnp.int32)
    thr = thr_row[:, :1]
    need = K - cnt_row[:, :1]
    gt = bits > thr
    eq = bits == thr
    eq_rank = jnp.cumsum(eq.astype(jnp.int32), axis=1)
    sel = gt | (eq & (eq_rank <= need))
    order = jnp.argsort(jnp.where(sel, 0, 1), axis=1, stable=True)
    idx = order[:, :K]
    wts = jnp.take_along_axis(probsT, idx, axis=1)
    return idx.astype(jnp.int32), wts


def kernel(inputs, router_w, W1, b1, ln_g, ln_b, W2, b2):
    B, S, _ = inputs.shape
    x = inputs.reshape(N, D)
    logits, probsT = _router_probs(x, router_w)
    thr_row, cnt_row = _thresholds(probsT)
    idx, wts = _compact_sc(probsT, thr_row, cnt_row)
    idx_flat = idx.reshape(-1)
    xg = _gather_sc(x, idx_flat)
    out01 = _ffn(xg, W1, b1, ln_g, ln_b, W2, b2, wts)
    outg2 = out01.reshape(2 * N, D // 2)
    res_il = _scatter_sc(outg2, idx_flat)
    results = res_il.reshape(N, D)
    return results.reshape(B, S, D), logits.reshape(B, S, E)


# speedup vs baseline: 2.4870x; 1.0969x over previous
import functools

import jax
import jax.numpy as jnp
from jax import lax
from jax.experimental import pallas as pl
from jax.experimental.pallas import tpu as pltpu
from jax.experimental.pallas import tpu_sc as plsc

_pallas_call = pl.pallas_call

N = 8192
D = 1024
F = 2048
E = 8
K = 1024
EPS_LN = 1e-05
MT = 512
MTILES = K // MT


_ROUTER_DTYPE = jnp.bfloat16


def _router_body(x_ref, rw_ref, logits_ref, probsT_ref):
    xb = x_ref[...].astype(_ROUTER_DTYPE)
    rw = rw_ref[...].astype(_ROUTER_DTYPE)
    logits_ref[...] = jnp.dot(xb, rw, preferred_element_type=jnp.float32)
    lT = lax.dot_general(rw, xb, (((0,), (1,)), ((), ())),
                         preferred_element_type=jnp.float32)
    m = jnp.max(lT, axis=0, keepdims=True)
    ex = jnp.exp(lT - m)
    probsT_ref[...] = ex / jnp.sum(ex, axis=0, keepdims=True)


def _thresh_body(probsT_ref, thr_ref, cnt_ref):
    bits = lax.bitcast_convert_type(probsT_ref[...], jnp.int32)

    def body(_, lohi):
        lo, hi = lohi
        mid = (lo + hi) // 2
        cnt = jnp.sum((bits >= mid).astype(jnp.int32), axis=1, keepdims=True)
        ge = cnt >= K
        return jnp.where(ge, mid, lo), jnp.where(ge, hi, mid)

    lo0 = jnp.zeros((E, 1), jnp.int32)
    hi0 = jnp.full((E, 1), 0x40000000, jnp.int32)
    thr, _ = lax.fori_loop(0, 31, body, (lo0, hi0))
    c = jnp.sum((bits >= (thr + 1)).astype(jnp.int32), axis=1, keepdims=True)
    thr_f = lax.bitcast_convert_type(thr, jnp.float32)
    thr_ref[...] = jnp.broadcast_to(thr_f, (E, 128))
    cnt_ref[...] = jnp.broadcast_to(c, (E, 128))


def _ffn_body(xg_ref, w1_ref, b1_ref, g_ref, bln_ref, w2_ref, b2_ref,
              wts_ref, out_ref):
    xi = xg_ref[...].astype(jnp.bfloat16)
    w1 = w1_ref[0].astype(jnp.bfloat16)
    h = jnp.dot(xi, w1, preferred_element_type=jnp.float32) + b1_ref[0]
    h = 0.5 * h * (1.0 + lax.erf(h * 0.7071067811865476))
    mu = jnp.mean(h, axis=-1, keepdims=True)
    var = jnp.mean(jnp.square(h - mu), axis=-1, keepdims=True)
    h = (h - mu) * lax.rsqrt(var + EPS_LN) * g_ref[0] + bln_ref[0]
    w2 = w2_ref[0].astype(jnp.bfloat16)
    out = jnp.dot(h.astype(jnp.bfloat16), w2,
                  preferred_element_type=jnp.float32) + b2_ref[0]
    wl = wts_ref[0]
    rows = lax.broadcasted_iota(jnp.int32, (MT, MT), 0)
    cols = lax.broadcasted_iota(jnp.int32, (MT, MT), 1)
    wcol = jnp.sum(jnp.where(rows == cols, wl, 0.0), axis=1, keepdims=True)
    out = out * wcol
    out_ref[0] = out[:, :D // 2]
    out_ref[1] = out[:, D // 2:]


def _router_probs(x, router_w):
    TB = 1024
    return _pallas_call(
        _router_body,
        grid=(N // TB,),
        in_specs=[pl.BlockSpec((TB, D), lambda i: (i, 0)),
                  pl.BlockSpec((D, E), lambda i: (0, 0))],
        out_specs=[pl.BlockSpec((TB, E), lambda i: (i, 0)),
                   pl.BlockSpec((E, TB), lambda i: (0, i))],
        out_shape=[jax.ShapeDtypeStruct((N, E), jnp.float32),
                   jax.ShapeDtypeStruct((E, N), jnp.float32)],
    )(x, router_w)


def _thresholds(probsT):
    return _pallas_call(
        _thresh_body,
        in_specs=[pl.BlockSpec((E, N), lambda: (0, 0))],
        out_specs=[pl.BlockSpec((E, 128), lambda: (0, 0)),
                   pl.BlockSpec((E, 128), lambda: (0, 0))],
        out_shape=[jax.ShapeDtypeStruct((E, 128), jnp.float32),
                   jax.ShapeDtypeStruct((E, 128), jnp.int32)],
    )(probsT)


def _ffn(xg, W1, b1, ln_g, ln_b, W2, b2, wts):
    b1r = b1.reshape(E, 1, F)
    gr = ln_g.reshape(E, 1, F)
    br = ln_b.reshape(E, 1, F)
    b2r = b2.reshape(E, 1, D)
    wtsr = wts.reshape(E, 1, K)
    return _pallas_call(
        _ffn_body,
        grid=(E, MTILES),
        in_specs=[pl.BlockSpec((MT, D), lambda e, m: (e * MTILES + m, 0)),
                  pl.BlockSpec((1, D, F), lambda e, m: (e, 0, 0)),
                  pl.BlockSpec((1, 1, F), lambda e, m: (e, 0, 0)),
                  pl.BlockSpec((1, 1, F), lambda e, m: (e, 0, 0)),
                  pl.BlockSpec((1, 1, F), lambda e, m: (e, 0, 0)),
                  pl.BlockSpec((1, F, D), lambda e, m: (e, 0, 0)),
                  pl.BlockSpec((1, 1, D), lambda e, m: (e, 0, 0)),
                  pl.BlockSpec((1, 1, MT), lambda e, m: (e, 0, m))],
        out_specs=pl.BlockSpec((2, MT, D // 2),
                               lambda e, m: (0, e * MTILES + m, 0)),
        out_shape=jax.ShapeDtypeStruct((2, N, D // 2), jnp.float32),
    )(xg, W1, b1r, gr, br, W2, b2r, wtsr)


_SC_MESH = plsc.VectorSubcoreMesh(core_axis_name="c", subcore_axis_name="s")


def _compact_body(probsT_hbm, thr_hbm, cnt_hbm, idx_out, wts_out,
                  pcol, thrv, cntv, idxv, wtsv):
    c = lax.axis_index("c")
    s = lax.axis_index("s")
    wid = s * 2 + c

    @pl.when(wid < E)
    def _():
        e = wid
        pltpu.sync_copy(probsT_hbm.at[e], pcol)
        pltpu.sync_copy(thr_hbm.at[e], thrv)
        pltpu.sync_copy(cnt_hbm.at[e], cntv)
        thr_vec = thrv[pl.ds(0, 16)]
        need_vec = K - cntv[pl.ds(0, 16)]

        def body(i, carry):
            off, eqseen = carry
            v = pcol[pl.ds(i * 16, 16)]
            gt = v > thr_vec
            eq = v == thr_vec
            eqc = plsc.cumsum(eq.astype(jnp.int32))
            sel = gt | (eq & ((eqc + eqseen) <= need_vec))
            selc = plsc.cumsum(sel.astype(jnp.int32))
            pos = jnp.where(sel, off + selc - 1, 0)
            tok = lax.iota(jnp.int32, 16) + i * 16
            plsc.store_scatter(idxv, [pos], tok, mask=sel)
            plsc.store_scatter(wtsv, [pos], v, mask=sel)
            nsel = jnp.sum(sel.astype(jnp.int32))
            neq = jnp.sum(eq.astype(jnp.int32))
            return off + nsel, eqseen + neq

        lax.fori_loop(0, N // 16, body, (jnp.int32(0), jnp.int32(0)))
        pltpu.sync_copy(idxv.at[pl.ds(0, K)], idx_out.at[e])
        pltpu.sync_copy(wtsv.at[pl.ds(0, K)], wts_out.at[e])


@functools.partial(
    pl.kernel, mesh=_SC_MESH,
    compiler_params=pltpu.CompilerParams(needs_layout_passes=False),
    out_type=[jax.ShapeDtypeStruct((E, K), jnp.int32),
              jax.ShapeDtypeStruct((E, K), jnp.float32)],
    scratch_types=[pltpu.VMEM((N,), jnp.float32),
                   pltpu.VMEM((128,), jnp.float32),
                   pltpu.VMEM((128,), jnp.int32),
                   pltpu.VMEM((K + 16,), jnp.int32),
                   pltpu.VMEM((K + 16,), jnp.float32)])
def _compact_sc(probsT_hbm, thr_hbm, cnt_hbm, idx_out, wts_out,
                pcol, thrv, cntv, idxv, wtsv):
    _compact_body(probsT_hbm, thr_hbm, cnt_hbm, idx_out, wts_out,
                  pcol, thrv, cntv, idxv, wtsv)


_GC = 64


def _gather_body(x_hbm, idx_hbm, xg_out, idxv, rowsv, sem):
    wid = lax.axis_index("s") * 2 + lax.axis_index("c")
    rows_per = N // 32

    def chunk(ci, _):
        b = wid * rows_per + ci * _GC
        pltpu.sync_copy(idx_hbm.at[pl.ds(b, _GC)], idxv)
        pltpu.async_copy(x_hbm.at[idxv], rowsv, sem).wait()
        pltpu.sync_copy(rowsv, xg_out.at[pl.ds(b, _GC)])
        return 0

    lax.fori_loop(0, rows_per // _GC, chunk, 0)


@functools.partial(
    pl.kernel, mesh=_SC_MESH,
    compiler_params=pltpu.CompilerParams(needs_layout_passes=False),
    out_type=jax.ShapeDtypeStruct((N, D), jnp.float32),
    scratch_types=[pltpu.VMEM((_GC,), jnp.int32),
                   pltpu.VMEM((_GC, D), jnp.float32),
                   pltpu.SemaphoreType.DMA])
def _gather_sc(x_hbm, idx_hbm, xg_out, idxv, rowsv, sem):
    _gather_body(x_hbm, idx_hbm, xg_out, idxv, rowsv, sem)


_PC = 128
_NPASS = (D // 2) // _PC


def _scatter_body(outg2_hbm, idx_hbm, res_out, idxv, srcv, zbuf, shared):
    h = lax.axis_index("c")
    w = lax.axis_index("s")
    zero16 = jnp.zeros((16,), jnp.float32)

    def zrow(r, _):
        for m in range(_PC // 16):
            zbuf[r, pl.ds(m * 16, 16)] = zero16
        return 0

    lax.fori_loop(0, 64, zrow, 0)
    for p in range(_NPASS):
        col = _PC * p
        for q in range(8):
            pltpu.sync_copy(zbuf, shared.at[pl.ds(w * 512 + q * 64, 64)])
        plsc.subcore_barrier()
        for e in range(E):
            base = e * K + w * 64
            pltpu.sync_copy(idx_hbm.at[pl.ds(base, 64)], idxv)
            pltpu.sync_copy(
                outg2_hbm.at[pl.ds(h * N + base, 64), pl.ds(col, _PC)], srcv)
            pltpu.sync_copy(srcv, shared.at[idxv], add=True)
        plsc.subcore_barrier()
        pltpu.sync_copy(
            shared.at[pl.ds(w * 512, 512)],
            res_out.at[pl.ds(w * 512, 512),
                       pl.ds(h * (D // 2) + col, _PC)])


@functools.partial(
    pl.kernel, mesh=_SC_MESH,
    compiler_params=pltpu.CompilerParams(needs_layout_passes=False),
    out_type=jax.ShapeDtypeStruct((N, D), jnp.float32),
    scratch_types=[pltpu.VMEM((64,), jnp.int32),
                   pltpu.VMEM((64, _PC), jnp.float32),
                   pltpu.VMEM((64, _PC), jnp.float32),
                   pltpu.VMEM_SHARED((N, _PC), jnp.float32)])
def _scatter_sc(outg2_hbm, idx_hbm, res_out, idxv, srcv, zbuf, shared):
    _scatter_body(outg2_hbm, idx_hbm, res_out, idxv, srcv, zbuf, shared)


def _select_glue(probsT, thr_row, cnt_row):
    bits = lax.bitcast_convert_type(probsT, jnp.int32)
    thr = thr_row[:, :1]
    need = K - cnt_row[:, :1]
    gt = bits > thr
    eq = bits == thr
    eq_rank = jnp.cumsum(eq.astype(jnp.int32), axis=1)
    sel = gt | (eq & (eq_rank <= need))
    order = jnp.argsort(jnp.where(sel, 0, 1), axis=1, stable=True)
    idx = order[:, :K]
    wts = jnp.take_along_axis(probsT, idx, axis=1)
    return idx.astype(jnp.int32), wts


def kernel(inputs, router_w, W1, b1, ln_g, ln_b, W2, b2):
    B, S, _ = inputs.shape
    x = inputs.reshape(N, D)
    logits, probsT = _router_probs(x, router_w)
    thr_row, cnt_row = _thresholds(probsT)
    idx, wts = _compact_sc(probsT, thr_row, cnt_row)
    idx_flat = idx.reshape(-1)
    xg = _gather_sc(x, idx_flat)
    out01 = _ffn(xg, W1, b1, ln_g, ln_b, W2, b2, wts)
    outg2 = out01.reshape(2 * N, D // 2)
    results = _scatter_sc(outg2, idx_flat)
    return results.reshape(B, S, D), logits.reshape(B, S, E)


# speedup vs baseline: 2.7622x; 1.1107x over previous
import functools

import jax
import jax.numpy as jnp
from jax import lax
from jax.experimental import pallas as pl
from jax.experimental.pallas import tpu as pltpu
from jax.experimental.pallas import tpu_sc as plsc

_pallas_call = pl.pallas_call

N = 8192
D = 1024
F = 2048
E = 8
K = 1024
EPS_LN = 1e-05
MT = 512
MTILES = K // MT


_ROUTER_DTYPE = jnp.bfloat16


def _router_body(x_ref, rw_ref, logits_ref, probsT_ref):
    xb = x_ref[...].astype(_ROUTER_DTYPE)
    rw = rw_ref[...].astype(_ROUTER_DTYPE)
    logits_ref[...] = jnp.dot(xb, rw, preferred_element_type=jnp.float32)
    lT = lax.dot_general(rw, xb, (((0,), (1,)), ((), ())),
                         preferred_element_type=jnp.float32)
    m = jnp.max(lT, axis=0, keepdims=True)
    ex = jnp.exp(lT - m)
    probsT_ref[...] = ex / jnp.sum(ex, axis=0, keepdims=True)


def _thresh_body(probsT_ref, thr_ref, cnt_ref):
    bits = lax.bitcast_convert_type(probsT_ref[...], jnp.int32)

    def body(_, lohi):
        lo, hi = lohi
        mid = (lo + hi) // 2
        cnt = jnp.sum((bits >= mid).astype(jnp.int32), axis=1, keepdims=True)
        ge = cnt >= K
        return jnp.where(ge, mid, lo), jnp.where(ge, hi, mid)

    lo0 = jnp.zeros((E, 1), jnp.int32)
    hi0 = jnp.full((E, 1), 0x40000000, jnp.int32)
    thr, _ = lax.fori_loop(0, 31, body, (lo0, hi0))
    c = jnp.sum((bits >= (thr + 1)).astype(jnp.int32), axis=1, keepdims=True)
    thr_f = lax.bitcast_convert_type(thr, jnp.float32)
    thr_ref[...] = jnp.broadcast_to(thr_f, (E, 128))
    cnt_ref[...] = jnp.broadcast_to(c, (E, 128))


def _ffn_body(xg_ref, w1_ref, b1_ref, g_ref, bln_ref, w2_ref, b2_ref,
              wts_ref, out_ref):
    xi = xg_ref[...].astype(jnp.bfloat16)
    w1 = w1_ref[0].astype(jnp.bfloat16)
    h = jnp.dot(xi, w1, preferred_element_type=jnp.float32) + b1_ref[0]
    h = 0.5 * h * (1.0 + lax.erf(h * 0.7071067811865476))
    mu = jnp.mean(h, axis=-1, keepdims=True)
    var = jnp.mean(jnp.square(h - mu), axis=-1, keepdims=True)
    h = (h - mu) * lax.rsqrt(var + EPS_LN) * g_ref[0] + bln_ref[0]
    w2 = w2_ref[0].astype(jnp.bfloat16)
    out = jnp.dot(h.astype(jnp.bfloat16), w2,
                  preferred_element_type=jnp.float32) + b2_ref[0]
    wl = wts_ref[0]
    rows = lax.broadcasted_iota(jnp.int32, (MT, MT), 0)
    cols = lax.broadcasted_iota(jnp.int32, (MT, MT), 1)
    wcol = jnp.sum(jnp.where(rows == cols, wl, 0.0), axis=1, keepdims=True)
    out = out * wcol
    out_ref[0] = out[:, :D // 2]
    out_ref[1] = out[:, D // 2:]


def _router_probs(x, router_w):
    TB = 1024
    return _pallas_call(
        _router_body,
        grid=(N // TB,),
        in_specs=[pl.BlockSpec((TB, D), lambda i: (i, 0)),
                  pl.BlockSpec((D, E), lambda i: (0, 0))],
        out_specs=[pl.BlockSpec((TB, E), lambda i: (i, 0)),
                   pl.BlockSpec((E, TB), lambda i: (0, i))],
        out_shape=[jax.ShapeDtypeStruct((N, E), jnp.float32),
                   jax.ShapeDtypeStruct((E, N), jnp.float32)],
    )(x, router_w)


def _thresholds(probsT):
    return _pallas_call(
        _thresh_body,
        in_specs=[pl.BlockSpec((E, N), lambda: (0, 0))],
        out_specs=[pl.BlockSpec((E, 128), lambda: (0, 0)),
                   pl.BlockSpec((E, 128), lambda: (0, 0))],
        out_shape=[jax.ShapeDtypeStruct((E, 128), jnp.float32),
                   jax.ShapeDtypeStruct((E, 128), jnp.int32)],
    )(probsT)


def _ffn(xg, W1, b1, ln_g, ln_b, W2, b2, wts):
    b1r = b1.reshape(E, 1, F)
    gr = ln_g.reshape(E, 1, F)
    br = ln_b.reshape(E, 1, F)
    b2r = b2.reshape(E, 1, D)
    wtsr = wts.reshape(E, 1, K)
    return _pallas_call(
        _ffn_body,
        grid=(E, MTILES),
        in_specs=[pl.BlockSpec((MT, D), lambda e, m: (e * MTILES + m, 0)),
                  pl.BlockSpec((1, D, F), lambda e, m: (e, 0, 0)),
                  pl.BlockSpec((1, 1, F), lambda e, m: (e, 0, 0)),
                  pl.BlockSpec((1, 1, F), lambda e, m: (e, 0, 0)),
                  pl.BlockSpec((1, 1, F), lambda e, m: (e, 0, 0)),
                  pl.BlockSpec((1, F, D), lambda e, m: (e, 0, 0)),
                  pl.BlockSpec((1, 1, D), lambda e, m: (e, 0, 0)),
                  pl.BlockSpec((1, 1, MT), lambda e, m: (e, 0, m))],
        out_specs=pl.BlockSpec((2, MT, D // 2),
                               lambda e, m: (0, e * MTILES + m, 0)),
        out_shape=jax.ShapeDtypeStruct((2, N, D // 2), jnp.float32),
    )(xg, W1, b1r, gr, br, W2, b2r, wtsr)


_SC_MESH = plsc.VectorSubcoreMesh(core_axis_name="c", subcore_axis_name="s")


def _compact_body(probsT_hbm, thr_hbm, cnt_hbm, idx_out, wts_out,
                  pcol, thrv, cntv, idxv, wtsv):
    c = lax.axis_index("c")
    s = lax.axis_index("s")
    wid = s * 2 + c

    @pl.when(wid < E)
    def _():
        e = wid
        pltpu.sync_copy(probsT_hbm.at[e], pcol)
        pltpu.sync_copy(thr_hbm.at[e], thrv)
        pltpu.sync_copy(cnt_hbm.at[e], cntv)
        thr_vec = thrv[pl.ds(0, 16)]
        need_vec = K - cntv[pl.ds(0, 16)]

        def body(i, carry):
            off, eqseen = carry
            v = pcol[pl.ds(i * 16, 16)]
            gt = v > thr_vec
            eq = v == thr_vec
            eqc = plsc.cumsum(eq.astype(jnp.int32))
            sel = gt | (eq & ((eqc + eqseen) <= need_vec))
            selc = plsc.cumsum(sel.astype(jnp.int32))
            pos = jnp.where(sel, off + selc - 1, 0)
            tok = lax.iota(jnp.int32, 16) + i * 16
            plsc.store_scatter(idxv, [pos], tok, mask=sel)
            plsc.store_scatter(wtsv, [pos], v, mask=sel)
            nsel = jnp.sum(sel.astype(jnp.int32))
            neq = jnp.sum(eq.astype(jnp.int32))
            return off + nsel, eqseen + neq

        lax.fori_loop(0, N // 16, body, (jnp.int32(0), jnp.int32(0)))
        pltpu.sync_copy(idxv.at[pl.ds(0, K)], idx_out.at[e])
        pltpu.sync_copy(wtsv.at[pl.ds(0, K)], wts_out.at[e])


@functools.partial(
    pl.kernel, mesh=_SC_MESH,
    compiler_params=pltpu.CompilerParams(needs_layout_passes=False),
    out_type=[jax.ShapeDtypeStruct((E, K), jnp.int32),
              jax.ShapeDtypeStruct((E, K), jnp.float32)],
    scratch_types=[pltpu.VMEM((N,), jnp.float32),
                   pltpu.VMEM((128,), jnp.float32),
                   pltpu.VMEM((128,), jnp.int32),
                   pltpu.VMEM((K + 16,), jnp.int32),
                   pltpu.VMEM((K + 16,), jnp.float32)])
def _compact_sc(probsT_hbm, thr_hbm, cnt_hbm, idx_out, wts_out,
                pcol, thrv, cntv, idxv, wtsv):
    _compact_body(probsT_hbm, thr_hbm, cnt_hbm, idx_out, wts_out,
                  pcol, thrv, cntv, idxv, wtsv)


_GC = 64


def _gather_body(x_hbm, idx_hbm, xg_out, idxv, rowsv, sem):
    wid = lax.axis_index("s") * 2 + lax.axis_index("c")
    rows_per = N // 32

    def chunk(ci, _):
        b = wid * rows_per + ci * _GC
        pltpu.sync_copy(idx_hbm.at[pl.ds(b, _GC)], idxv)
        pltpu.async_copy(x_hbm.at[idxv], rowsv, sem).wait()
        pltpu.sync_copy(rowsv, xg_out.at[pl.ds(b, _GC)])
        return 0

    lax.fori_loop(0, rows_per // _GC, chunk, 0)


@functools.partial(
    pl.kernel, mesh=_SC_MESH,
    compiler_params=pltpu.CompilerParams(needs_layout_passes=False),
    out_type=jax.ShapeDtypeStruct((N, D), jnp.float32),
    scratch_types=[pltpu.VMEM((_GC,), jnp.int32),
                   pltpu.VMEM((_GC, D), jnp.float32),
                   pltpu.SemaphoreType.DMA])
def _gather_sc(x_hbm, idx_hbm, xg_out, idxv, rowsv, sem):
    _gather_body(x_hbm, idx_hbm, xg_out, idxv, rowsv, sem)


_PC = 128
_NPASS = (D // 2) // _PC


def _scatter_body(outg2_hbm, idx_hbm, res_out, idxv, src0, src1, zbuf,
                  shared, ldsem0, ldsem1, addsem, zsem, rbsem):
    h = lax.axis_index("c")
    w = lax.axis_index("s")
    zero16 = jnp.zeros((16,), jnp.float32)

    def zrow(r, _):
        for m in range(_PC // 16):
            zbuf[r, pl.ds(m * 16, 16)] = zero16
        return 0

    lax.fori_loop(0, 128, zrow, 0)
    for e in range(E):
        pltpu.sync_copy(idx_hbm.at[pl.ds(e * K + w * 64, 64)], idxv.at[e])
    srcs = (src0, src1)
    ldsems = (ldsem0, ldsem1)
    for p in range(_NPASS):
        col = _PC * p
        if p > 0:
            pltpu.make_async_copy(shared.at[pl.ds(w * 512, 512)],
                                  res_out.at[pl.ds(w * 512, 512),
                                             pl.ds(0, _PC)], rbsem).wait()
        for q in range(4):
            pltpu.async_copy(zbuf, shared.at[pl.ds(w * 512 + q * 128, 128)],
                             zsem)
        for q in range(4):
            pltpu.make_async_copy(zbuf,
                                  shared.at[pl.ds(w * 512, 128)],
                                  zsem).wait()
        plsc.subcore_barrier()

        def load(e):
            base = e * K + w * 64
            return pltpu.async_copy(
                outg2_hbm.at[pl.ds(h * N + base, 64), pl.ds(col, _PC)],
                srcs[e % 2], ldsems[e % 2])

        load(0)
        for e in range(E):
            if e + 1 < E:
                load(e + 1)
            pltpu.make_async_copy(
                outg2_hbm.at[pl.ds(0, 64), pl.ds(col, _PC)],
                srcs[e % 2], ldsems[e % 2]).wait()
            pltpu.async_copy(srcs[e % 2], shared.at[idxv.at[e]],
                             addsem, add=True).wait()
        plsc.subcore_barrier()
        pltpu.async_copy(
            shared.at[pl.ds(w * 512, 512)],
            res_out.at[pl.ds(w * 512, 512),
                       pl.ds(h * (D // 2) + col, _PC)], rbsem)
    pltpu.make_async_copy(shared.at[pl.ds(w * 512, 512)],
                          res_out.at[pl.ds(w * 512, 512),
                                     pl.ds(0, _PC)], rbsem).wait()


@functools.partial(
    pl.kernel, mesh=_SC_MESH,
    compiler_params=pltpu.CompilerParams(needs_layout_passes=False),
    out_type=jax.ShapeDtypeStruct((N, D), jnp.float32),
    scratch_types=[pltpu.VMEM((E, 64), jnp.int32),
                   pltpu.VMEM((64, _PC), jnp.float32),
                   pltpu.VMEM((64, _PC), jnp.float32),
                   pltpu.VMEM((128, _PC), jnp.float32),
                   pltpu.VMEM_SHARED((N, _PC), jnp.float32),
                   pltpu.SemaphoreType.DMA,
                   pltpu.SemaphoreType.DMA,
                   pltpu.SemaphoreType.DMA,
                   pltpu.SemaphoreType.DMA,
                   pltpu.SemaphoreType.DMA])
def _scatter_sc(outg2_hbm, idx_hbm, res_out, idxv, src0, src1, zbuf,
                shared, ldsem0, ldsem1, addsem, zsem, rbsem):
    _scatter_body(outg2_hbm, idx_hbm, res_out, idxv, src0, src1, zbuf,
                  shared, ldsem0, ldsem1, addsem, zsem, rbsem)


def _select_glue(probsT, thr_row, cnt_row):
    bits = lax.bitcast_convert_type(probsT, jnp.int32)
    thr = thr_row[:, :1]
    need = K - cnt_row[:, :1]
    gt = bits > thr
    eq = bits == thr
    eq_rank = jnp.cumsum(eq.astype(jnp.int32), axis=1)
    sel = gt | (eq & (eq_rank <= need))
    order = jnp.argsort(jnp.where(sel, 0, 1), axis=1, stable=True)
    idx = order[:, :K]
    wts = jnp.take_along_axis(probsT, idx, axis=1)
    return idx.astype(jnp.int32), wts


def kernel(inputs, router_w, W1, b1, ln_g, ln_b, W2, b2):
    B, S, _ = inputs.shape
    x = inputs.reshape(N, D)
    logits, probsT = _router_probs(x, router_w)
    thr_row, cnt_row = _thresholds(probsT)
    idx, wts = _compact_sc(probsT, thr_row, cnt_row)
    idx_flat = idx.reshape(-1)
    xg = _gather_sc(x, idx_flat)
    out01 = _ffn(xg, W1, b1, ln_g, ln_b, W2, b2, wts)
    outg2 = out01.reshape(2 * N, D // 2)
    results = _scatter_sc(outg2, idx_flat)
    return results.reshape(B, S, D), logits.reshape(B, S, E)


# speedup vs baseline: 2.7644x; 1.0008x over previous
import functools

import jax
import jax.numpy as jnp
from jax import lax
from jax.experimental import pallas as pl
from jax.experimental.pallas import tpu as pltpu
from jax.experimental.pallas import tpu_sc as plsc

_pallas_call = pl.pallas_call

N = 8192
D = 1024
F = 2048
E = 8
K = 1024
EPS_LN = 1e-05
MT = 512
MTILES = K // MT


_ROUTER_DTYPE = jnp.bfloat16


def _router_body(x_ref, rw_ref, logits_ref, probsT_ref):
    xb = x_ref[...].astype(_ROUTER_DTYPE)
    rw = rw_ref[...].astype(_ROUTER_DTYPE)
    logits_ref[...] = jnp.dot(xb, rw, preferred_element_type=jnp.float32)
    lT = lax.dot_general(rw, xb, (((0,), (1,)), ((), ())),
                         preferred_element_type=jnp.float32)
    m = jnp.max(lT, axis=0, keepdims=True)
    ex = jnp.exp(lT - m)
    probsT_ref[...] = ex / jnp.sum(ex, axis=0, keepdims=True)


def _thresh_body(probsT_ref, thr_ref, cnt_ref):
    bits = lax.bitcast_convert_type(probsT_ref[...], jnp.int32)

    def body(_, lohi):
        lo, hi = lohi
        mid = (lo + hi) // 2
        cnt = jnp.sum((bits >= mid).astype(jnp.int32), axis=1, keepdims=True)
        ge = cnt >= K
        return jnp.where(ge, mid, lo), jnp.where(ge, hi, mid)

    lo0 = jnp.zeros((E, 1), jnp.int32)
    hi0 = jnp.full((E, 1), 0x40000000, jnp.int32)
    thr, _ = lax.fori_loop(0, 31, body, (lo0, hi0))
    c = jnp.sum((bits >= (thr + 1)).astype(jnp.int32), axis=1, keepdims=True)
    thr_f = lax.bitcast_convert_type(thr, jnp.float32)
    thr_ref[...] = jnp.broadcast_to(thr_f, (E, 128))
    cnt_ref[...] = jnp.broadcast_to(c, (E, 128))


def _ffn_body(xg_ref, w1_ref, b1_ref, g_ref, bln_ref, w2_ref, b2_ref,
              wts_ref, out_ref):
    xi = xg_ref[...].astype(jnp.bfloat16)
    w1 = w1_ref[0].astype(jnp.bfloat16)
    h = jnp.dot(xi, w1, preferred_element_type=jnp.float32) + b1_ref[0]
    h = 0.5 * h * (1.0 + lax.erf(h * 0.7071067811865476))
    mu = jnp.mean(h, axis=-1, keepdims=True)
    var = jnp.mean(jnp.square(h - mu), axis=-1, keepdims=True)
    h = (h - mu) * lax.rsqrt(var + EPS_LN) * g_ref[0] + bln_ref[0]
    w2 = w2_ref[0].astype(jnp.bfloat16)
    out = jnp.dot(h.astype(jnp.bfloat16), w2,
                  preferred_element_type=jnp.float32) + b2_ref[0]
    wl = wts_ref[0]
    rows = lax.broadcasted_iota(jnp.int32, (MT, MT), 0)
    cols = lax.broadcasted_iota(jnp.int32, (MT, MT), 1)
    wcol = jnp.sum(jnp.where(rows == cols, wl, 0.0), axis=1, keepdims=True)
    out = out * wcol
    out_ref[0] = out[:, :D // 2]
    out_ref[1] = out[:, D // 2:]


def _router_probs(x, router_w):
    TB = 1024
    return _pallas_call(
        _router_body,
        grid=(N // TB,),
        in_specs=[pl.BlockSpec((TB, D), lambda i: (i, 0)),
                  pl.BlockSpec((D, E), lambda i: (0, 0))],
        out_specs=[pl.BlockSpec((TB, E), lambda i: (i, 0)),
                   pl.BlockSpec((E, TB), lambda i: (0, i))],
        out_shape=[jax.ShapeDtypeStruct((N, E), jnp.float32),
                   jax.ShapeDtypeStruct((E, N), jnp.float32)],
    )(x, router_w)


def _thresholds(probsT):
    return _pallas_call(
        _thresh_body,
        in_specs=[pl.BlockSpec((E, N), lambda: (0, 0))],
        out_specs=[pl.BlockSpec((E, 128), lambda: (0, 0)),
                   pl.BlockSpec((E, 128), lambda: (0, 0))],
        out_shape=[jax.ShapeDtypeStruct((E, 128), jnp.float32),
                   jax.ShapeDtypeStruct((E, 128), jnp.int32)],
    )(probsT)


def _ffn(xg, W1, b1, ln_g, ln_b, W2, b2, wts):
    b1r = b1.reshape(E, 1, F)
    gr = ln_g.reshape(E, 1, F)
    br = ln_b.reshape(E, 1, F)
    b2r = b2.reshape(E, 1, D)
    wtsr = wts.reshape(E, 1, K)
    return _pallas_call(
        _ffn_body,
        grid=(E, MTILES),
        in_specs=[pl.BlockSpec((MT, D), lambda e, m: (e * MTILES + m, 0)),
                  pl.BlockSpec((1, D, F), lambda e, m: (e, 0, 0)),
                  pl.BlockSpec((1, 1, F), lambda e, m: (e, 0, 0)),
                  pl.BlockSpec((1, 1, F), lambda e, m: (e, 0, 0)),
                  pl.BlockSpec((1, 1, F), lambda e, m: (e, 0, 0)),
                  pl.BlockSpec((1, F, D), lambda e, m: (e, 0, 0)),
                  pl.BlockSpec((1, 1, D), lambda e, m: (e, 0, 0)),
                  pl.BlockSpec((1, 1, MT), lambda e, m: (e, 0, m))],
        out_specs=pl.BlockSpec((2, MT, D // 2),
                               lambda e, m: (0, e * MTILES + m, 0)),
        out_shape=jax.ShapeDtypeStruct((2, N, D // 2), jnp.float32),
    )(xg, W1, b1r, gr, br, W2, b2r, wtsr)


_SC_MESH = plsc.VectorSubcoreMesh(core_axis_name="c", subcore_axis_name="s")


def _compact_body(probsT_hbm, thr_hbm, cnt_hbm, idx_out, wts_out,
                  pcol, thrv, cntv, idxv, wtsv):
    c = lax.axis_index("c")
    s = lax.axis_index("s")
    wid = s * 2 + c

    @pl.when(wid < E)
    def _():
        e = wid
        pltpu.sync_copy(probsT_hbm.at[e], pcol)
        pltpu.sync_copy(thr_hbm.at[e], thrv)
        pltpu.sync_copy(cnt_hbm.at[e], cntv)
        thr_vec = thrv[pl.ds(0, 16)]
        need_vec = K - cntv[pl.ds(0, 16)]

        def body(i, carry):
            off, eqseen = carry
            v = pcol[pl.ds(i * 16, 16)]
            gt = v > thr_vec
            eq = v == thr_vec
            eqc = plsc.cumsum(eq.astype(jnp.int32))
            sel = gt | (eq & ((eqc + eqseen) <= need_vec))
            selc = plsc.cumsum(sel.astype(jnp.int32))
            pos = jnp.where(sel, off + selc - 1, 0)
            tok = lax.iota(jnp.int32, 16) + i * 16
            plsc.store_scatter(idxv, [pos], tok, mask=sel)
            plsc.store_scatter(wtsv, [pos], v, mask=sel)
            nsel = jnp.sum(sel.astype(jnp.int32))
            neq = jnp.sum(eq.astype(jnp.int32))
            return off + nsel, eqseen + neq

        lax.fori_loop(0, N // 16, body, (jnp.int32(0), jnp.int32(0)))
        pltpu.sync_copy(idxv.at[pl.ds(0, K)], idx_out.at[e])
        pltpu.sync_copy(wtsv.at[pl.ds(0, K)], wts_out.at[e])


@functools.partial(
    pl.kernel, mesh=_SC_MESH,
    compiler_params=pltpu.CompilerParams(needs_layout_passes=False),
    out_type=[jax.ShapeDtypeStruct((E, K), jnp.int32),
              jax.ShapeDtypeStruct((E, K), jnp.float32)],
    scratch_types=[pltpu.VMEM((N,), jnp.float32),
                   pltpu.VMEM((128,), jnp.float32),
                   pltpu.VMEM((128,), jnp.int32),
                   pltpu.VMEM((K + 16,), jnp.int32),
                   pltpu.VMEM((K + 16,), jnp.float32)])
def _compact_sc(probsT_hbm, thr_hbm, cnt_hbm, idx_out, wts_out,
                pcol, thrv, cntv, idxv, wtsv):
    _compact_body(probsT_hbm, thr_hbm, cnt_hbm, idx_out, wts_out,
                  pcol, thrv, cntv, idxv, wtsv)


_GC = 32
_NGC = (N // 32) // _GC


def _gather_body(x_hbm, idx_hbm, xg_out, idxv, rows0, rows1,
                 gsem0, gsem1, wsem0, wsem1):
    wid = lax.axis_index("s") * 2 + lax.axis_index("c")
    rows_per = N // 32
    base = wid * rows_per
    pltpu.sync_copy(idx_hbm.at[pl.ds(base, rows_per)], idxv)
    bufs = (rows0, rows1)
    gsems = (gsem0, gsem1)
    wsems = (wsem0, wsem1)

    def gather(c):
        return pltpu.async_copy(
            x_hbm.at[idxv.at[pl.ds(c * _GC, _GC)]], bufs[c % 2],
            gsems[c % 2])

    gds = [gather(0)]
    wds = [None] * _NGC
    for c in range(_NGC):
        if c + 1 < _NGC:
            if c - 1 >= 0:
                wds[c - 1].wait()
            gds.append(gather(c + 1))
        gds[c].wait()
        wds[c] = pltpu.async_copy(
            bufs[c % 2], xg_out.at[pl.ds(base + c * _GC, _GC)],
            wsems[c % 2])
    wds[_NGC - 2].wait()
    wds[_NGC - 1].wait()


@functools.partial(
    pl.kernel, mesh=_SC_MESH,
    compiler_params=pltpu.CompilerParams(needs_layout_passes=False),
    out_type=jax.ShapeDtypeStruct((N, D), jnp.float32),
    scratch_types=[pltpu.VMEM((N // 32,), jnp.int32),
                   pltpu.VMEM((_GC, D), jnp.float32),
                   pltpu.VMEM((_GC, D), jnp.float32),
                   pltpu.SemaphoreType.DMA,
                   pltpu.SemaphoreType.DMA,
                   pltpu.SemaphoreType.DMA,
                   pltpu.SemaphoreType.DMA])
def _gather_sc(x_hbm, idx_hbm, xg_out, idxv, rows0, rows1,
               gsem0, gsem1, wsem0, wsem1):
    _gather_body(x_hbm, idx_hbm, xg_out, idxv, rows0, rows1,
                 gsem0, gsem1, wsem0, wsem1)


_PC = 128
_NPASS = (D // 2) // _PC


def _scatter_body(outg2_hbm, idx_hbm, res_out, idxv, src0, src1, zbuf,
                  shared, ldsem0, ldsem1, addsem, zsem, rbsem):
    h = lax.axis_index("c")
    w = lax.axis_index("s")
    zero16 = jnp.zeros((16,), jnp.float32)

    def zrow(r, _):
        for m in range(_PC // 16):
            zbuf[r, pl.ds(m * 16, 16)] = zero16
        return 0

    lax.fori_loop(0, 128, zrow, 0)
    for e in range(E):
        pltpu.sync_copy(idx_hbm.at[pl.ds(e * K + w * 64, 64)], idxv.at[e])
    srcs = (src0, src1)
    ldsems = (ldsem0, ldsem1)
    for p in range(_NPASS):
        col = _PC * p
        if p > 0:
            pltpu.make_async_copy(shared.at[pl.ds(w * 512, 512)],
                                  res_out.at[pl.ds(w * 512, 512),
                                             pl.ds(0, _PC)], rbsem).wait()
        for q in range(4):
            pltpu.async_copy(zbuf, shared.at[pl.ds(w * 512 + q * 128, 128)],
                             zsem)
        for q in range(4):
            pltpu.make_async_copy(zbuf,
                                  shared.at[pl.ds(w * 512, 128)],
                                  zsem).wait()
        plsc.subcore_barrier()

        def load(e):
            base = e * K + w * 64
            return pltpu.async_copy(
                outg2_hbm.at[pl.ds(h * N + base, 64), pl.ds(col, _PC)],
                srcs[e % 2], ldsems[e % 2])

        load(0)
        for e in range(E):
            if e + 1 < E:
                load(e + 1)
            pltpu.make_async_copy(
                outg2_hbm.at[pl.ds(0, 64), pl.ds(col, _PC)],
                srcs[e % 2], ldsems[e % 2]).wait()
            pltpu.async_copy(srcs[e % 2], shared.at[idxv.at[e]],
                             addsem, add=True).wait()
        plsc.subcore_barrier()
        pltpu.async_copy(
            shared.at[pl.ds(w * 512, 512)],
            res_out.at[pl.ds(w * 512, 512),
                       pl.ds(h * (D // 2) + col, _PC)], rbsem)
    pltpu.make_async_copy(shared.at[pl.ds(w * 512, 512)],
                          res_out.at[pl.ds(w * 512, 512),
                                     pl.ds(0, _PC)], rbsem).wait()


@functools.partial(
    pl.kernel, mesh=_SC_MESH,
    compiler_params=pltpu.CompilerParams(needs_layout_passes=False),
    out_type=jax.ShapeDtypeStruct((N, D), jnp.float32),
    scratch_types=[pltpu.VMEM((E, 64), jnp.int32),
                   pltpu.VMEM((64, _PC), jnp.float32),
                   pltpu.VMEM((64, _PC), jnp.float32),
                   pltpu.VMEM((128, _PC), jnp.float32),
                   pltpu.VMEM_SHARED((N, _PC), jnp.float32),
                   pltpu.SemaphoreType.DMA,
                   pltpu.SemaphoreType.DMA,
                   pltpu.SemaphoreType.DMA,
                   pltpu.SemaphoreType.DMA,
                   pltpu.SemaphoreType.DMA])
def _scatter_sc(outg2_hbm, idx_hbm, res_out, idxv, src0, src1, zbuf,
                shared, ldsem0, ldsem1, addsem, zsem, rbsem):
    _scatter_body(outg2_hbm, idx_hbm, res_out, idxv, src0, src1, zbuf,
                  shared, ldsem0, ldsem1, addsem, zsem, rbsem)


def _select_glue(probsT, thr_row, cnt_row):
    bits = lax.bitcast_convert_type(probsT, jnp.int32)
    thr = thr_row[:, :1]
    need = K - cnt_row[:, :1]
    gt = bits > thr
    eq = bits == thr
    eq_rank = jnp.cumsum(eq.astype(jnp.int32), axis=1)
    sel = gt | (eq & (eq_rank <= need))
    order = jnp.argsort(jnp.where(sel, 0, 1), axis=1, stable=True)
    idx = order[:, :K]
    wts = jnp.take_along_axis(probsT, idx, axis=1)
    return idx.astype(jnp.int32), wts


def kernel(inputs, router_w, W1, b1, ln_g, ln_b, W2, b2):
    B, S, _ = inputs.shape
    x = inputs.reshape(N, D)
    logits, probsT = _router_probs(x, router_w)
    thr_row, cnt_row = _thresholds(probsT)
    idx, wts = _compact_sc(probsT, thr_row, cnt_row)
    idx_flat = idx.reshape(-1)
    xg = _gather_sc(x, idx_flat)
    out01 = _ffn(xg, W1, b1, ln_g, ln_b, W2, b2, wts)
    outg2 = out01.reshape(2 * N, D // 2)
    results = _scatter_sc(outg2, idx_flat)
    return results.reshape(B, S, D), logits.reshape(B, S, E)


# speedup vs baseline: 2.7668x; 1.0009x over previous
import functools

import jax
import jax.numpy as jnp
from jax import lax
from jax.experimental import pallas as pl
from jax.experimental.pallas import tpu as pltpu
from jax.experimental.pallas import tpu_sc as plsc

_pallas_call = pl.pallas_call

N = 8192
D = 1024
F = 2048
E = 8
K = 1024
EPS_LN = 1e-05
MT = 512
MTILES = K // MT


_ROUTER_DTYPE = jnp.bfloat16


def _router_body(x_ref, rw_ref, logits_ref, probsT_ref):
    xb = x_ref[...].astype(_ROUTER_DTYPE)
    rw = rw_ref[...].astype(_ROUTER_DTYPE)
    logits_ref[...] = jnp.dot(xb, rw, preferred_element_type=jnp.float32)
    lT = lax.dot_general(rw, xb, (((0,), (1,)), ((), ())),
                         preferred_element_type=jnp.float32)
    m = jnp.max(lT, axis=0, keepdims=True)
    ex = jnp.exp(lT - m)
    probsT_ref[...] = ex / jnp.sum(ex, axis=0, keepdims=True)


def _thresh_body(probsT_ref, thr_ref, cnt_ref):
    bits = lax.bitcast_convert_type(probsT_ref[...], jnp.int32)

    def body(_, lohi):
        lo, hi = lohi
        mid = (lo + hi) // 2
        cnt = jnp.sum((bits >= mid).astype(jnp.int32), axis=1, keepdims=True)
        ge = cnt >= K
        return jnp.where(ge, mid, lo), jnp.where(ge, hi, mid)

    lo0 = jnp.zeros((E, 1), jnp.int32)
    hi0 = jnp.full((E, 1), 0x40000000, jnp.int32)
    thr, _ = lax.fori_loop(0, 31, body, (lo0, hi0))
    c = jnp.sum((bits >= (thr + 1)).astype(jnp.int32), axis=1, keepdims=True)
    thr_f = lax.bitcast_convert_type(thr, jnp.float32)
    thr_ref[...] = jnp.broadcast_to(thr_f, (E, 128))
    cnt_ref[...] = jnp.broadcast_to(c, (E, 128))


def _ffn_body(xg_ref, w1_ref, b1_ref, g_ref, bln_ref, w2_ref, b2_ref,
              wts_ref, out_ref):
    xi = xg_ref[...].astype(jnp.bfloat16)
    w1 = w1_ref[0].astype(jnp.bfloat16)
    h = jnp.dot(xi, w1, preferred_element_type=jnp.float32) + b1_ref[0]
    h = 0.5 * h * (1.0 + lax.erf(h * 0.7071067811865476))
    mu = jnp.mean(h, axis=-1, keepdims=True)
    var = jnp.mean(jnp.square(h - mu), axis=-1, keepdims=True)
    h = (h - mu) * lax.rsqrt(var + EPS_LN) * g_ref[0] + bln_ref[0]
    w2 = w2_ref[0].astype(jnp.bfloat16)
    out = jnp.dot(h.astype(jnp.bfloat16), w2,
                  preferred_element_type=jnp.float32) + b2_ref[0]
    wl = wts_ref[0]
    rows = lax.broadcasted_iota(jnp.int32, (MT, MT), 0)
    cols = lax.broadcasted_iota(jnp.int32, (MT, MT), 1)
    wcol = jnp.sum(jnp.where(rows == cols, wl, 0.0), axis=1, keepdims=True)
    out = out * wcol
    out_ref[0] = out[:, :D // 2]
    out_ref[1] = out[:, D // 2:]


def _router_probs(x, router_w):
    TB = 1024
    return _pallas_call(
        _router_body,
        grid=(N // TB,),
        in_specs=[pl.BlockSpec((TB, D), lambda i: (i, 0)),
                  pl.BlockSpec((D, E), lambda i: (0, 0))],
        out_specs=[pl.BlockSpec((TB, E), lambda i: (i, 0)),
                   pl.BlockSpec((E, TB), lambda i: (0, i))],
        out_shape=[jax.ShapeDtypeStruct((N, E), jnp.float32),
                   jax.ShapeDtypeStruct((E, N), jnp.float32)],
    )(x, router_w)


def _thresholds(probsT):
    return _pallas_call(
        _thresh_body,
        in_specs=[pl.BlockSpec((E, N), lambda: (0, 0))],
        out_specs=[pl.BlockSpec((E, 128), lambda: (0, 0)),
                   pl.BlockSpec((E, 128), lambda: (0, 0))],
        out_shape=[jax.ShapeDtypeStruct((E, 128), jnp.float32),
                   jax.ShapeDtypeStruct((E, 128), jnp.int32)],
    )(probsT)


def _ffn(xg, W1, b1, ln_g, ln_b, W2, b2, wts):
    b1r = b1.reshape(E, 1, F)
    gr = ln_g.reshape(E, 1, F)
    br = ln_b.reshape(E, 1, F)
    b2r = b2.reshape(E, 1, D)
    wtsr = wts.reshape(E, 1, K)
    return _pallas_call(
        _ffn_body,
        grid=(E, MTILES),
        in_specs=[pl.BlockSpec((MT, D), lambda e, m: (e * MTILES + m, 0)),
                  pl.BlockSpec((1, D, F), lambda e, m: (e, 0, 0)),
                  pl.BlockSpec((1, 1, F), lambda e, m: (e, 0, 0)),
                  pl.BlockSpec((1, 1, F), lambda e, m: (e, 0, 0)),
                  pl.BlockSpec((1, 1, F), lambda e, m: (e, 0, 0)),
                  pl.BlockSpec((1, F, D), lambda e, m: (e, 0, 0)),
                  pl.BlockSpec((1, 1, D), lambda e, m: (e, 0, 0)),
                  pl.BlockSpec((1, 1, MT), lambda e, m: (e, 0, m))],
        out_specs=pl.BlockSpec((2, MT, D // 2),
                               lambda e, m: (0, e * MTILES + m, 0)),
        out_shape=jax.ShapeDtypeStruct((2, N, D // 2), jnp.float32),
    )(xg, W1, b1r, gr, br, W2, b2r, wtsr)


_SC_MESH = plsc.VectorSubcoreMesh(core_axis_name="c", subcore_axis_name="s")


def _compact_body(probsT_hbm, thr_hbm, cnt_hbm, idx_out, wts_out,
                  pcol, thrv, cntv, idxv, wtsv):
    c = lax.axis_index("c")
    s = lax.axis_index("s")
    wid = s * 2 + c

    @pl.when(wid < E)
    def _():
        e = wid
        pltpu.sync_copy(probsT_hbm.at[e], pcol)
        pltpu.sync_copy(thr_hbm.at[e], thrv)
        pltpu.sync_copy(cnt_hbm.at[e], cntv)
        thr_vec = thrv[pl.ds(0, 16)]
        need_vec = K - cntv[pl.ds(0, 16)]

        def body(i, carry):
            off, eqseen = carry
            v = pcol[pl.ds(i * 16, 16)]
            gt = v > thr_vec
            eq = v == thr_vec
            eqc = plsc.cumsum(eq.astype(jnp.int32))
            sel = gt | (eq & ((eqc + eqseen) <= need_vec))
            selc = plsc.cumsum(sel.astype(jnp.int32))
            pos = jnp.where(sel, off + selc - 1, 0)
            tok = lax.iota(jnp.int32, 16) + i * 16
            plsc.store_scatter(idxv, [pos], tok, mask=sel)
            plsc.store_scatter(wtsv, [pos], v, mask=sel)
            nsel = jnp.sum(sel.astype(jnp.int32))
            neq = jnp.sum(eq.astype(jnp.int32))
            return off + nsel, eqseen + neq

        lax.fori_loop(0, N // 16, body, (jnp.int32(0), jnp.int32(0)))
        pltpu.sync_copy(idxv.at[pl.ds(0, K)], idx_out.at[e])
        pltpu.sync_copy(wtsv.at[pl.ds(0, K)], wts_out.at[e])


@functools.partial(
    pl.kernel, mesh=_SC_MESH,
    compiler_params=pltpu.CompilerParams(needs_layout_passes=False),
    out_type=[jax.ShapeDtypeStruct((E, K), jnp.int32),
              jax.ShapeDtypeStruct((E, K), jnp.float32)],
    scratch_types=[pltpu.VMEM((N,), jnp.float32),
                   pltpu.VMEM((128,), jnp.float32),
                   pltpu.VMEM((128,), jnp.int32),
                   pltpu.VMEM((K + 16,), jnp.int32),
                   pltpu.VMEM((K + 16,), jnp.float32)])
def _compact_sc(probsT_hbm, thr_hbm, cnt_hbm, idx_out, wts_out,
                pcol, thrv, cntv, idxv, wtsv):
    _compact_body(probsT_hbm, thr_hbm, cnt_hbm, idx_out, wts_out,
                  pcol, thrv, cntv, idxv, wtsv)


_GC = 32
_NGC = (N // 32) // _GC


def _gather_body(x_hbm, idx_hbm, xg_out, idxv, rows0, rows1,
                 gsem0, gsem1, wsem0, wsem1):
    wid = lax.axis_index("s") * 2 + lax.axis_index("c")
    rows_per = N // 32
    base = wid * rows_per
    pltpu.sync_copy(idx_hbm.at[pl.ds(base, rows_per)], idxv)
    bufs = (rows0, rows1)
    gsems = (gsem0, gsem1)
    wsems = (wsem0, wsem1)

    def gather(c):
        return pltpu.async_copy(
            x_hbm.at[idxv.at[pl.ds(c * _GC, _GC)]], bufs[c % 2],
            gsems[c % 2])

    gds = [gather(0)]
    wds = [None] * _NGC
    for c in range(_NGC):
        if c + 1 < _NGC:
            if c - 1 >= 0:
                wds[c - 1].wait()
            gds.append(gather(c + 1))
        gds[c].wait()
        wds[c] = pltpu.async_copy(
            bufs[c % 2], xg_out.at[pl.ds(base + c * _GC, _GC)],
            wsems[c % 2])
    wds[_NGC - 2].wait()
    wds[_NGC - 1].wait()


@functools.partial(
    pl.kernel, mesh=_SC_MESH,
    compiler_params=pltpu.CompilerParams(needs_layout_passes=False),
    out_type=jax.ShapeDtypeStruct((N, D), jnp.float32),
    scratch_types=[pltpu.VMEM((N // 32,), jnp.int32),
                   pltpu.VMEM((_GC, D), jnp.float32),
                   pltpu.VMEM((_GC, D), jnp.float32),
                   pltpu.SemaphoreType.DMA,
                   pltpu.SemaphoreType.DMA,
                   pltpu.SemaphoreType.DMA,
                   pltpu.SemaphoreType.DMA])
def _gather_sc(x_hbm, idx_hbm, xg_out, idxv, rows0, rows1,
               gsem0, gsem1, wsem0, wsem1):
    _gather_body(x_hbm, idx_hbm, xg_out, idxv, rows0, rows1,
                 gsem0, gsem1, wsem0, wsem1)


_PC = 128
_NPASS = (D // 2) // _PC


def _scatter_body(outg2_hbm, idx_hbm, res_out, idxv, src0, src1, zbuf,
                  shared, ldsem0, ldsem1, addsem, zsem, rbsem):
    h = lax.axis_index("c")
    w = lax.axis_index("s")
    zero16 = jnp.zeros((16,), jnp.float32)

    def zrow(r, _):
        for m in range(_PC // 16):
            zbuf[r, pl.ds(m * 16, 16)] = zero16
        return 0

    lax.fori_loop(0, 128, zrow, 0)
    for e in range(E):
        pltpu.sync_copy(idx_hbm.at[pl.ds(e * K + w * 64, 64)], idxv.at[e])
    srcs = (src0, src1)
    ldsems = (ldsem0, ldsem1)
    for p in range(_NPASS):
        col = _PC * p
        if p > 0:
            pltpu.make_async_copy(shared.at[pl.ds(w * 512, 512)],
                                  res_out.at[pl.ds(w * 512, 512),
                                             pl.ds(0, _PC)], rbsem).wait()
        for q in range(4):
            pltpu.async_copy(zbuf, shared.at[pl.ds(w * 512 + q * 128, 128)],
                             zsem)
        for q in range(4):
            pltpu.make_async_copy(zbuf,
                                  shared.at[pl.ds(w * 512, 128)],
                                  zsem).wait()
        plsc.subcore_barrier()

        def load(e):
            base = e * K + w * 64
            return pltpu.async_copy(
                outg2_hbm.at[pl.ds(h * N + base, 64), pl.ds(col, _PC)],
                srcs[e % 2], ldsems[e % 2])

        load(0)
        for e in range(E):
            if e + 1 < E:
                load(e + 1)
            pltpu.make_async_copy(
                outg2_hbm.at[pl.ds(0, 64), pl.ds(col, _PC)],
                srcs[e % 2], ldsems[e % 2]).wait()
            pltpu.async_copy(srcs[e % 2], shared.at[idxv.at[e]],
                             addsem, add=True).wait()
        plsc.subcore_barrier()
        pltpu.async_copy(
            shared.at[pl.ds(w * 512, 512)],
            res_out.at[pl.ds(w * 512, 512),
                       pl.ds(h * (D // 2) + col, _PC)], rbsem)
    pltpu.make_async_copy(shared.at[pl.ds(w * 512, 512)],
                          res_out.at[pl.ds(w * 512, 512),
                                     pl.ds(0, _PC)], rbsem).wait()


@functools.partial(
    pl.kernel, mesh=_SC_MESH,
    compiler_params=pltpu.CompilerParams(needs_layout_passes=False),
    out_type=jax.ShapeDtypeStruct((N, D), jnp.float32),
    scratch_types=[pltpu.VMEM((E, 64), jnp.int32),
                   pltpu.VMEM((64, _PC), jnp.float32),
                   pltpu.VMEM((64, _PC), jnp.float32),
                   pltpu.VMEM((128, _PC), jnp.float32),
                   pltpu.VMEM_SHARED((N, _PC), jnp.float32),
                   pltpu.SemaphoreType.DMA,
                   pltpu.SemaphoreType.DMA,
                   pltpu.SemaphoreType.DMA,
                   pltpu.SemaphoreType.DMA,
                   pltpu.SemaphoreType.DMA])
def _scatter_sc(outg2_hbm, idx_hbm, res_out, idxv, src0, src1, zbuf,
                shared, ldsem0, ldsem1, addsem, zsem, rbsem):
    _scatter_body(outg2_hbm, idx_hbm, res_out, idxv, src0, src1, zbuf,
                  shared, ldsem0, ldsem1, addsem, zsem, rbsem)


def kernel(inputs, router_w, W1, b1, ln_g, ln_b, W2, b2):
    B, S, _ = inputs.shape
    x = inputs.reshape(N, D)
    logits, probsT = _router_probs(x, router_w)
    thr_row, cnt_row = _thresholds(probsT)
    idx, wts = _compact_sc(probsT, thr_row, cnt_row)
    idx_flat = idx.reshape(-1)
    xg = _gather_sc(x, idx_flat)
    out01 = _ffn(xg, W1, b1, ln_g, ln_b, W2, b2, wts)
    outg2 = out01.reshape(2 * N, D // 2)
    results = _scatter_sc(outg2, idx_flat)
    return results.reshape(B, S, D), logits.reshape(B, S, E)
